# Initial kernel scaffold; baseline (speedup 1.0000x reference)
import jax
import jax.numpy as jnp
from jax.experimental import pallas as pl


def kernel(x, edge_index, Ws0, b0, Ws1, b1, Ws2, b2, Wf, bf):
    raise NotImplementedError("write your pallas kernel here")


# speedup vs baseline: 9.0620x; 9.0620x over previous
import functools

import jax
import jax.numpy as jnp
from jax import lax
from jax.experimental import pallas as pl
from jax.experimental.pallas import tpu as pltpu
from jax.experimental.pallas import tpu_sc as plsc

N = 10000
E = 320000
D = 128
NC = 2
NS = 16
NW = NC * NS
EPW = E // NW
CH = 80
NCH = EPW // CH
NPAD = 10240
RPT = NPAD // NS
RPTE = RPT

_MESH = plsc.VectorSubcoreMesh(core_axis_name="c", subcore_axis_name="s")


@functools.partial(
    pl.kernel,
    out_type=jax.ShapeDtypeStruct((NC, NPAD), jnp.float32),
    mesh=_MESH,
    scratch_types=[
        pltpu.VMEM((NCH, CH), jnp.int32),
        pltpu.VMEM((CH,), jnp.float32),
        pltpu.VMEM_SHARED((NPAD,), jnp.float32),
    ],
)
def _deg_kernel(dst_hbm, zero_hbm, out_hbm, dst_v, ones_v, acc_sh):
    cid = lax.axis_index("c")
    sid = lax.axis_index("s")
    wid = cid * NS + sid
    pltpu.sync_copy(dst_hbm.at[wid], dst_v)
    for i in range(CH // 16):
        ones_v[pl.ds(i * 16, 16)] = jnp.ones((16,), jnp.float32)
    pltpu.sync_copy(zero_hbm.at[pl.ds(sid * RPTE, RPTE)],
                    acc_sh.at[pl.ds(sid * RPTE, RPTE)])
    plsc.subcore_barrier()

    def chunk(j, carry):
        pltpu.sync_copy(ones_v, acc_sh.at[dst_v.at[j]], add=True)
        return carry

    lax.fori_loop(0, NCH, chunk, 0)
    plsc.subcore_barrier()
    pltpu.sync_copy(acc_sh.at[pl.ds(sid * RPTE, RPTE)],
                    out_hbm.at[cid, pl.ds(sid * RPTE, RPTE)])


@functools.partial(
    pl.kernel,
    out_type=jax.ShapeDtypeStruct((NC, NPAD, D), jnp.float32),
    mesh=_MESH,
    scratch_types=[
        pltpu.VMEM((NCH, CH), jnp.int32),
        pltpu.VMEM((NCH, CH), jnp.int32),
        pltpu.VMEM((CH, D), jnp.float32),
        pltpu.VMEM_SHARED((NPAD, D), jnp.float32),
        pltpu.SemaphoreType.DMA,
    ],
)
def _prop_kernel(t_hbm, src_hbm, dst_hbm, zero_hbm, out_hbm,
                 src_v, dst_v, rows_v, acc_sh, sem):
    cid = lax.axis_index("c")
    sid = lax.axis_index("s")
    wid = cid * NS + sid
    pltpu.sync_copy(src_hbm.at[wid], src_v)
    pltpu.sync_copy(dst_hbm.at[wid], dst_v)
    pltpu.sync_copy(zero_hbm.at[pl.ds(sid * RPT, RPT)],
                    acc_sh.at[pl.ds(sid * RPT, RPT)])
    plsc.subcore_barrier()

    def chunk(j, carry):
        pltpu.async_copy(t_hbm.at[src_v.at[j]], rows_v, sem).wait()
        pltpu.sync_copy(rows_v, acc_sh.at[dst_v.at[j]], add=True)
        return carry

    lax.fori_loop(0, NCH, chunk, 0)
    plsc.subcore_barrier()
    pltpu.sync_copy(acc_sh.at[pl.ds(sid * RPT, RPT)],
                    out_hbm.at[cid, pl.ds(sid * RPT, RPT)])


def _init_body(deg_ref, x_ref, dis_ref, t0_ref):
    deg = deg_ref[0] + deg_ref[1]
    deg = deg[:N]
    dis = jnp.where(deg > 0, lax.rsqrt(deg), jnp.float32(0))
    dis_ref[...] = dis
    t0_ref[...] = x_ref[...] * dis


def _init_call(deg2, x):
    return pl.pallas_call(
        _init_body,
        out_shape=(jax.ShapeDtypeStruct((N, 1), jnp.float32),
                   jax.ShapeDtypeStruct((N, D), jnp.float32)),
    )(deg2, x)


def _scale_body(p_ref, dis_ref, t_ref):
    d2 = dis_ref[...] * dis_ref[...]
    t_ref[...] = (p_ref[0, :N] + p_ref[1, :N]) * d2


def _scale_call(p, dis):
    return pl.pallas_call(
        _scale_body,
        out_shape=jax.ShapeDtypeStruct((N, D), jnp.float32),
    )(p, dis)


NB = 2000


def _combine_mid_body(h_ref, p1_ref, p2_ref, p3_ref, dis_ref, w_ref, b_ref,
                      ho_ref, to_ref):
    dis = dis_ref[...]
    acc = jnp.dot(h_ref[...], w_ref[0], preferred_element_type=jnp.float32)
    for k, pr in ((1, p1_ref), (2, p2_ref), (3, p3_ref)):
        s = (pr[0] + pr[1]) * dis
        acc = acc + jnp.dot(s, w_ref[k], preferred_element_type=jnp.float32)
    h = jnp.maximum(acc + b_ref[...], 0.0)
    ho_ref[...] = h
    to_ref[...] = h * dis


def _combine_last_body(h_ref, p1_ref, p2_ref, p3_ref, dis_ref, w_ref, b_ref,
                       wf_ref, bf_ref, z_ref):
    dis = dis_ref[...]
    acc = jnp.dot(h_ref[...], w_ref[0], preferred_element_type=jnp.float32)
    for k, pr in ((1, p1_ref), (2, p2_ref), (3, p3_ref)):
        s = (pr[0] + pr[1]) * dis
        acc = acc + jnp.dot(s, w_ref[k], preferred_element_type=jnp.float32)
    h = jnp.maximum(acc + b_ref[...], 0.0)
    z_ref[...] = jnp.dot(h, wf_ref[...], preferred_element_type=jnp.float32) \
        + bf_ref[...]


def _row_blocked_specs():
    row = pl.BlockSpec((NB, D), lambda i: (i, 0))
    par = pl.BlockSpec((NC, NB, D), lambda i: (0, i, 0))
    disb = pl.BlockSpec((NB, 1), lambda i: (i, 0))
    w = pl.BlockSpec((4, D, D), lambda i: (0, 0, 0))
    b = pl.BlockSpec((1, D), lambda i: (0, 0))
    return row, par, disb, w, b


def _combine_mid(h, p1, p2, p3, dis, Ws, b):
    row, par, disb, w, bsp = _row_blocked_specs()
    return pl.pallas_call(
        _combine_mid_body,
        grid=(N // NB,),
        in_specs=[row, par, par, par, disb, w, bsp],
        out_specs=(row, row),
        out_shape=(jax.ShapeDtypeStruct((N, D), jnp.float32),
                   jax.ShapeDtypeStruct((N, D), jnp.float32)),
    )(h, p1, p2, p3, dis, Ws, b)


def _combine_last(h, p1, p2, p3, dis, Ws, b, Wf, bf):
    row, par, disb, w, bsp = _row_blocked_specs()
    wf = pl.BlockSpec((D, D), lambda i: (0, 0))
    return pl.pallas_call(
        _combine_last_body,
        grid=(N // NB,),
        in_specs=[row, par, par, par, disb, w, bsp, wf, bsp],
        out_specs=row,
        out_shape=jax.ShapeDtypeStruct((N, D), jnp.float32),
    )(h, p1, p2, p3, dis, Ws, b, Wf, bf)


def kernel(x, edge_index, Ws0, b0, Ws1, b1, Ws2, b2, Wf, bf):
    src3 = edge_index[0].reshape(NW, NCH, CH)
    dst3 = edge_index[1].reshape(NW, NCH, CH)
    zero_nd = jnp.zeros((NPAD, D), jnp.float32)
    zero_np = jnp.zeros((NPAD,), jnp.float32)

    deg2 = _deg_kernel(dst3, zero_np)
    dis, t = _init_call(deg2.reshape(NC, NPAD, 1), x)

    h = x
    for li, (Ws, b) in enumerate(((Ws0, b0), (Ws1, b1), (Ws2, b2))):
        b2d = b.reshape(1, D)
        p1 = _prop_kernel(t, src3, dst3, zero_nd)
        t = _scale_call(p1, dis)
        p2 = _prop_kernel(t, src3, dst3, zero_nd)
        t = _scale_call(p2, dis)
        p3 = _prop_kernel(t, src3, dst3, zero_nd)
        if li < 2:
            h, t = _combine_mid(h, p1, p2, p3, dis, Ws, b2d)
        else:
            z = _combine_last(h, p1, p2, p3, dis, Ws, b2d,
                              Wf, bf.reshape(1, D))
    return z


# speedup vs baseline: 16.1903x; 1.7866x over previous
import functools

import jax
import jax.numpy as jnp
from jax import lax
from jax.experimental import pallas as pl
from jax.experimental.pallas import tpu as pltpu
from jax.experimental.pallas import tpu_sc as plsc

N = 10000
E = 320000
D = 128
NC = 2
NS = 16
NW = NC * NS
EPW = E // NW
CH = 128
NCHP = 80
EPAD = NW * NCHP * CH
NPADE = EPAD - E
NPAD = 10240
RPT = NPAD // NS
RPTE = RPT

_MESH = plsc.VectorSubcoreMesh(core_axis_name="c", subcore_axis_name="s")


def _unpack_chunk(packed_v, j, src_buf, dst_buf):
    for i in range(CH // 16):
        w = packed_v[j, pl.ds(i * 16, 16)]
        if src_buf is not None:
            src_buf[pl.ds(i * 16, 16)] = jnp.bitwise_and(w, 0xFFFF)
        dst_buf[pl.ds(i * 16, 16)] = lax.shift_right_logical(w, 16)


@functools.partial(
    pl.kernel,
    out_type=jax.ShapeDtypeStruct((NC, NPAD), jnp.float32),
    mesh=_MESH,
    scratch_types=[
        pltpu.VMEM((NCHP, CH), jnp.int32),
        pltpu.VMEM((CH,), jnp.int32),
        pltpu.VMEM((CH,), jnp.float32),
        pltpu.VMEM_SHARED((NPAD,), jnp.float32),
    ],
)
def _deg_kernel(edges_hbm, zero_hbm, out_hbm, packed_v, dst_b, ones_v, acc_sh):
    cid = lax.axis_index("c")
    sid = lax.axis_index("s")
    wid = cid * NS + sid
    pltpu.sync_copy(edges_hbm.at[wid], packed_v)
    for i in range(CH // 16):
        ones_v[pl.ds(i * 16, 16)] = jnp.ones((16,), jnp.float32)
    pltpu.sync_copy(zero_hbm.at[pl.ds(sid * RPTE, RPTE)],
                    acc_sh.at[pl.ds(sid * RPTE, RPTE)])
    plsc.subcore_barrier()

    def chunk(j, carry):
        _unpack_chunk(packed_v, j, None, dst_b)
        pltpu.sync_copy(ones_v, acc_sh.at[dst_b], add=True)
        return carry

    lax.fori_loop(0, NCHP, chunk, 0)
    plsc.subcore_barrier()
    pltpu.sync_copy(acc_sh.at[pl.ds(sid * RPTE, RPTE)],
                    out_hbm.at[cid, pl.ds(sid * RPTE, RPTE)])


@functools.partial(
    pl.kernel,
    out_type=jax.ShapeDtypeStruct((NC, NPAD, D), jnp.float32),
    mesh=_MESH,
    scratch_types=[
        pltpu.VMEM((NCHP, CH), jnp.int32),
        pltpu.VMEM((CH,), jnp.int32),
        pltpu.VMEM((CH,), jnp.int32),
        pltpu.VMEM((CH,), jnp.int32),
        pltpu.VMEM((CH,), jnp.int32),
        pltpu.VMEM((CH, D), jnp.float32),
        pltpu.VMEM((CH, D), jnp.float32),
        pltpu.VMEM_SHARED((NPAD, D), jnp.float32),
        pltpu.SemaphoreType.DMA,
        pltpu.SemaphoreType.DMA,
    ],
)
def _prop_kernel(t_hbm, edges_hbm, zero_hbm, out_hbm,
                 packed_v, src0_b, dst0_b, src1_b, dst1_b,
                 rows0_v, rows1_v, acc_sh, sem0, sem1):
    cid = lax.axis_index("c")
    sid = lax.axis_index("s")
    wid = cid * NS + sid
    pltpu.sync_copy(edges_hbm.at[wid], packed_v)
    pltpu.sync_copy(zero_hbm.at[pl.ds(sid * RPT, RPT)],
                    acc_sh.at[pl.ds(sid * RPT, RPT)])
    plsc.subcore_barrier()

    _unpack_chunk(packed_v, 0, src0_b, dst0_b)
    pltpu.async_copy(t_hbm.at[src0_b], rows0_v, sem0)
    _unpack_chunk(packed_v, 1, src1_b, dst1_b)
    pltpu.async_copy(t_hbm.at[src1_b], rows1_v, sem1)

    def pair(i, carry):
        j = 2 * i
        pltpu.make_async_copy(t_hbm.at[src0_b], rows0_v, sem0).wait()
        pltpu.sync_copy(rows0_v, acc_sh.at[dst0_b], add=True)

        @pl.when(j + 2 < NCHP)
        def _():
            _unpack_chunk(packed_v, j + 2, src0_b, dst0_b)
            pltpu.async_copy(t_hbm.at[src0_b], rows0_v, sem0)

        pltpu.make_async_copy(t_hbm.at[src1_b], rows1_v, sem1).wait()
        pltpu.sync_copy(rows1_v, acc_sh.at[dst1_b], add=True)

        @pl.when(j + 3 < NCHP)
        def _():
            _unpack_chunk(packed_v, j + 3, src1_b, dst1_b)
            pltpu.async_copy(t_hbm.at[src1_b], rows1_v, sem1)

        return carry

    lax.fori_loop(0, NCHP // 2, pair, 0)
    plsc.subcore_barrier()
    pltpu.sync_copy(acc_sh.at[pl.ds(sid * RPT, RPT)],
                    out_hbm.at[cid, pl.ds(sid * RPT, RPT)])


def _init_body(deg_ref, x_ref, dis_ref, t0_ref):
    deg = deg_ref[0] + deg_ref[1]
    deg = deg[:N]
    dis = jnp.where(deg > 0, lax.rsqrt(deg), jnp.float32(0))
    dis_ref[...] = dis
    t0_ref[...] = x_ref[...] * dis


def _init_call(deg2, x):
    return pl.pallas_call(
        _init_body,
        out_shape=(jax.ShapeDtypeStruct((N, 1), jnp.float32),
                   jax.ShapeDtypeStruct((N, D), jnp.float32)),
    )(deg2, x)


def _scale_body(p_ref, dis_ref, t_ref):
    d2 = dis_ref[...] * dis_ref[...]
    t_ref[...] = (p_ref[0, :N] + p_ref[1, :N]) * d2


def _scale_call(p, dis):
    return pl.pallas_call(
        _scale_body,
        out_shape=jax.ShapeDtypeStruct((N, D), jnp.float32),
    )(p, dis)


NB = 2000


def _combine_mid_body(h_ref, p1_ref, p2_ref, p3_ref, dis_ref, w_ref, b_ref,
                      ho_ref, to_ref):
    dis = dis_ref[...]
    acc = jnp.dot(h_ref[...], w_ref[0], preferred_element_type=jnp.float32)
    for k, pr in ((1, p1_ref), (2, p2_ref), (3, p3_ref)):
        s = (pr[0] + pr[1]) * dis
        acc = acc + jnp.dot(s, w_ref[k], preferred_element_type=jnp.float32)
    h = jnp.maximum(acc + b_ref[...], 0.0)
    ho_ref[...] = h
    to_ref[...] = h * dis


def _combine_last_body(h_ref, p1_ref, p2_ref, p3_ref, dis_ref, w_ref, b_ref,
                       wf_ref, bf_ref, z_ref):
    dis = dis_ref[...]
    acc = jnp.dot(h_ref[...], w_ref[0], preferred_element_type=jnp.float32)
    for k, pr in ((1, p1_ref), (2, p2_ref), (3, p3_ref)):
        s = (pr[0] + pr[1]) * dis
        acc = acc + jnp.dot(s, w_ref[k], preferred_element_type=jnp.float32)
    h = jnp.maximum(acc + b_ref[...], 0.0)
    z_ref[...] = jnp.dot(h, wf_ref[...], preferred_element_type=jnp.float32) \
        + bf_ref[...]


def _row_blocked_specs():
    row = pl.BlockSpec((NB, D), lambda i: (i, 0))
    par = pl.BlockSpec((NC, NB, D), lambda i: (0, i, 0))
    disb = pl.BlockSpec((NB, 1), lambda i: (i, 0))
    w = pl.BlockSpec((4, D, D), lambda i: (0, 0, 0))
    b = pl.BlockSpec((1, D), lambda i: (0, 0))
    return row, par, disb, w, b


def _combine_mid(h, p1, p2, p3, dis, Ws, b):
    row, par, disb, w, bsp = _row_blocked_specs()
    return pl.pallas_call(
        _combine_mid_body,
        grid=(N // NB,),
        in_specs=[row, par, par, par, disb, w, bsp],
        out_specs=(row, row),
        out_shape=(jax.ShapeDtypeStruct((N, D), jnp.float32),
                   jax.ShapeDtypeStruct((N, D), jnp.float32)),
    )(h, p1, p2, p3, dis, Ws, b)


def _combine_last(h, p1, p2, p3, dis, Ws, b, Wf, bf):
    row, par, disb, w, bsp = _row_blocked_specs()
    wf = pl.BlockSpec((D, D), lambda i: (0, 0))
    return pl.pallas_call(
        _combine_last_body,
        grid=(N // NB,),
        in_specs=[row, par, par, par, disb, w, bsp, wf, bsp],
        out_specs=row,
        out_shape=jax.ShapeDtypeStruct((N, D), jnp.float32),
    )(h, p1, p2, p3, dis, Ws, b, Wf, bf)


def kernel(x, edge_index, Ws0, b0, Ws1, b1, Ws2, b2, Wf, bf):
    ar = jnp.arange(NPADE, dtype=jnp.int32)
    src_p = jnp.concatenate([edge_index[0], ar % N])
    dst_p = jnp.concatenate([edge_index[1], N + ar % (NPAD - N)])
    packed = (src_p | (dst_p << 16)).reshape(NW, NCHP, CH)
    zero_nd = jnp.zeros((NPAD, D), jnp.float32)
    zero_np = jnp.zeros((NPAD,), jnp.float32)

    deg2 = _deg_kernel(packed, zero_np)
    dis, t = _init_call(deg2.reshape(NC, NPAD, 1), x)

    h = x
    for li, (Ws, b) in enumerate(((Ws0, b0), (Ws1, b1), (Ws2, b2))):
        b2d = b.reshape(1, D)
        p1 = _prop_kernel(t, packed, zero_nd)
        t = _scale_call(p1, dis)
        p2 = _prop_kernel(t, packed, zero_nd)
        t = _scale_call(p2, dis)
        p3 = _prop_kernel(t, packed, zero_nd)
        if li < 2:
            h, t = _combine_mid(h, p1, p2, p3, dis, Ws, b2d)
        else:
            z = _combine_last(h, p1, p2, p3, dis, Ws, b2d,
                              Wf, bf.reshape(1, D))
    return z


# speedup vs baseline: 16.4047x; 1.0132x over previous
import functools

import jax
import jax.numpy as jnp
from jax import lax
from jax.experimental import pallas as pl
from jax.experimental.pallas import tpu as pltpu
from jax.experimental.pallas import tpu_sc as plsc

N = 10000
E = 320000
D = 128
NC = 2
NS = 16
NW = NC * NS
EPW = E // NW
CH = 128
NCHP = 80
EPAD = NW * NCHP * CH
NPADE = EPAD - E
NPAD = 10240
RPT = NPAD // NS
RPTE = RPT

_MESH = plsc.VectorSubcoreMesh(core_axis_name="c", subcore_axis_name="s")


def _unpack_chunk(packed_v, j, src_buf, dst_buf):
    for i in range(CH // 16):
        w = packed_v[j, pl.ds(i * 16, 16)]
        if src_buf is not None:
            src_buf[pl.ds(i * 16, 16)] = jnp.bitwise_and(w, 0xFFFF)
        dst_buf[pl.ds(i * 16, 16)] = lax.shift_right_logical(w, 16)


@functools.partial(
    pl.kernel,
    out_type=jax.ShapeDtypeStruct((NC, NPAD), jnp.float32),
    mesh=_MESH,
    scratch_types=[
        pltpu.VMEM((NCHP, CH), jnp.int32),
        pltpu.VMEM((CH,), jnp.int32),
        pltpu.VMEM((CH,), jnp.float32),
        pltpu.VMEM_SHARED((NPAD,), jnp.float32),
    ],
)
def _deg_kernel(edges_hbm, zero_hbm, out_hbm, packed_v, dst_b, ones_v, acc_sh):
    cid = lax.axis_index("c")
    sid = lax.axis_index("s")
    wid = cid * NS + sid
    pltpu.sync_copy(edges_hbm.at[wid], packed_v)
    for i in range(CH // 16):
        ones_v[pl.ds(i * 16, 16)] = jnp.ones((16,), jnp.float32)
    pltpu.sync_copy(zero_hbm.at[pl.ds(sid * RPTE, RPTE)],
                    acc_sh.at[pl.ds(sid * RPTE, RPTE)])
    plsc.subcore_barrier()

    def chunk(j, carry):
        _unpack_chunk(packed_v, j, None, dst_b)
        pltpu.sync_copy(ones_v, acc_sh.at[dst_b], add=True)
        return carry

    lax.fori_loop(0, NCHP, chunk, 0)
    plsc.subcore_barrier()
    pltpu.sync_copy(acc_sh.at[pl.ds(sid * RPTE, RPTE)],
                    out_hbm.at[cid, pl.ds(sid * RPTE, RPTE)])


@functools.partial(
    pl.kernel,
    out_type=jax.ShapeDtypeStruct((NC, NPAD, D), jnp.float32),
    mesh=_MESH,
    scratch_types=[
        pltpu.VMEM((NCHP, CH), jnp.int32),
        pltpu.VMEM((CH,), jnp.int32),
        pltpu.VMEM((CH,), jnp.int32),
        pltpu.VMEM((CH,), jnp.int32),
        pltpu.VMEM((CH,), jnp.int32),
        pltpu.VMEM((CH, D), jnp.float32),
        pltpu.VMEM((CH, D), jnp.float32),
        pltpu.VMEM_SHARED((NPAD, D), jnp.float32),
        pltpu.SemaphoreType.DMA,
        pltpu.SemaphoreType.DMA,
    ],
)
def _prop_kernel(t_hbm, edges_hbm, zero_hbm, out_hbm,
                 packed_v, src0_b, dst0_b, src1_b, dst1_b,
                 rows0_v, rows1_v, acc_sh, sem0, sem1):
    cid = lax.axis_index("c")
    sid = lax.axis_index("s")
    wid = cid * NS + sid
    pltpu.sync_copy(edges_hbm.at[wid], packed_v)

    _unpack_chunk(packed_v, 0, src0_b, dst0_b)
    pltpu.async_copy(t_hbm.at[src0_b], rows0_v, sem0)
    _unpack_chunk(packed_v, 1, src1_b, dst1_b)
    pltpu.async_copy(t_hbm.at[src1_b], rows1_v, sem1)
    pltpu.sync_copy(zero_hbm.at[pl.ds(sid * RPT, RPT)],
                    acc_sh.at[pl.ds(sid * RPT, RPT)])
    plsc.subcore_barrier()

    def pair(i, carry):
        j = 2 * i
        pltpu.make_async_copy(t_hbm.at[src0_b], rows0_v, sem0).wait()
        pltpu.sync_copy(rows0_v, acc_sh.at[dst0_b], add=True)

        @pl.when(j + 2 < NCHP)
        def _():
            _unpack_chunk(packed_v, j + 2, src0_b, dst0_b)
            pltpu.async_copy(t_hbm.at[src0_b], rows0_v, sem0)

        pltpu.make_async_copy(t_hbm.at[src1_b], rows1_v, sem1).wait()
        pltpu.sync_copy(rows1_v, acc_sh.at[dst1_b], add=True)

        @pl.when(j + 3 < NCHP)
        def _():
            _unpack_chunk(packed_v, j + 3, src1_b, dst1_b)
            pltpu.async_copy(t_hbm.at[src1_b], rows1_v, sem1)

        return carry

    lax.fori_loop(0, NCHP // 2, pair, 0)
    plsc.subcore_barrier()
    pltpu.sync_copy(acc_sh.at[pl.ds(sid * RPT, RPT)],
                    out_hbm.at[cid, pl.ds(sid * RPT, RPT)])


def _init_body(deg_ref, x_ref, dis_ref, t0_ref):
    deg = deg_ref[0] + deg_ref[1]
    deg = deg[:N]
    dis = jnp.where(deg > 0, lax.rsqrt(deg), jnp.float32(0))
    dis_ref[...] = dis
    t0_ref[...] = x_ref[...] * dis


def _init_call(deg2, x):
    return pl.pallas_call(
        _init_body,
        out_shape=(jax.ShapeDtypeStruct((N, 1), jnp.float32),
                   jax.ShapeDtypeStruct((N, D), jnp.float32)),
    )(deg2, x)


def _scale_body(p_ref, dis_ref, t_ref):
    d2 = dis_ref[...] * dis_ref[...]
    t_ref[...] = (p_ref[0] + p_ref[1]) * d2


def _scale_call(p, dis):
    return pl.pallas_call(
        _scale_body,
        grid=(N // NB,),
        in_specs=[pl.BlockSpec((NC, NB, D), lambda i: (0, i, 0)),
                  pl.BlockSpec((NB, 1), lambda i: (i, 0))],
        out_specs=pl.BlockSpec((NB, D), lambda i: (i, 0)),
        out_shape=jax.ShapeDtypeStruct((N, D), jnp.float32),
    )(p, dis)


NB = 2000


def _combine_mid_body(h_ref, p1_ref, p2_ref, p3_ref, dis_ref, w_ref, b_ref,
                      ho_ref, to_ref):
    dis = dis_ref[...]
    acc = jnp.dot(h_ref[...], w_ref[0], preferred_element_type=jnp.float32)
    for k, pr in ((1, p1_ref), (2, p2_ref), (3, p3_ref)):
        s = (pr[0] + pr[1]) * dis
        acc = acc + jnp.dot(s, w_ref[k], preferred_element_type=jnp.float32)
    h = jnp.maximum(acc + b_ref[...], 0.0)
    ho_ref[...] = h
    to_ref[...] = h * dis


def _combine_last_body(h_ref, p1_ref, p2_ref, p3_ref, dis_ref, w_ref, b_ref,
                       wf_ref, bf_ref, z_ref):
    dis = dis_ref[...]
    acc = jnp.dot(h_ref[...], w_ref[0], preferred_element_type=jnp.float32)
    for k, pr in ((1, p1_ref), (2, p2_ref), (3, p3_ref)):
        s = (pr[0] + pr[1]) * dis
        acc = acc + jnp.dot(s, w_ref[k], preferred_element_type=jnp.float32)
    h = jnp.maximum(acc + b_ref[...], 0.0)
    z_ref[...] = jnp.dot(h, wf_ref[...], preferred_element_type=jnp.float32) \
        + bf_ref[...]


def _row_blocked_specs():
    row = pl.BlockSpec((NB, D), lambda i: (i, 0))
    par = pl.BlockSpec((NC, NB, D), lambda i: (0, i, 0))
    disb = pl.BlockSpec((NB, 1), lambda i: (i, 0))
    w = pl.BlockSpec((4, D, D), lambda i: (0, 0, 0))
    b = pl.BlockSpec((1, D), lambda i: (0, 0))
    return row, par, disb, w, b


def _combine_mid(h, p1, p2, p3, dis, Ws, b):
    row, par, disb, w, bsp = _row_blocked_specs()
    return pl.pallas_call(
        _combine_mid_body,
        grid=(N // NB,),
        in_specs=[row, par, par, par, disb, w, bsp],
        out_specs=(row, row),
        out_shape=(jax.ShapeDtypeStruct((N, D), jnp.float32),
                   jax.ShapeDtypeStruct((N, D), jnp.float32)),
    )(h, p1, p2, p3, dis, Ws, b)


def _combine_last(h, p1, p2, p3, dis, Ws, b, Wf, bf):
    row, par, disb, w, bsp = _row_blocked_specs()
    wf = pl.BlockSpec((D, D), lambda i: (0, 0))
    return pl.pallas_call(
        _combine_last_body,
        grid=(N // NB,),
        in_specs=[row, par, par, par, disb, w, bsp, wf, bsp],
        out_specs=row,
        out_shape=jax.ShapeDtypeStruct((N, D), jnp.float32),
    )(h, p1, p2, p3, dis, Ws, b, Wf, bf)


def kernel(x, edge_index, Ws0, b0, Ws1, b1, Ws2, b2, Wf, bf):
    ar = jnp.arange(NPADE, dtype=jnp.int32)
    src_p = jnp.concatenate([edge_index[0], ar % N])
    dst_p = jnp.concatenate([edge_index[1], N + ar % (NPAD - N)])
    packed = (src_p | (dst_p << 16)).reshape(NW, NCHP, CH)
    zero_nd = jnp.zeros((NPAD, D), jnp.float32)
    zero_np = jnp.zeros((NPAD,), jnp.float32)

    deg2 = _deg_kernel(packed, zero_np)
    dis, t = _init_call(deg2.reshape(NC, NPAD, 1), x)

    h = x
    for li, (Ws, b) in enumerate(((Ws0, b0), (Ws1, b1), (Ws2, b2))):
        b2d = b.reshape(1, D)
        p1 = _prop_kernel(t, packed, zero_nd)
        t = _scale_call(p1, dis)
        p2 = _prop_kernel(t, packed, zero_nd)
        t = _scale_call(p2, dis)
        p3 = _prop_kernel(t, packed, zero_nd)
        if li < 2:
            h, t = _combine_mid(h, p1, p2, p3, dis, Ws, b2d)
        else:
            z = _combine_last(h, p1, p2, p3, dis, Ws, b2d,
                              Wf, bf.reshape(1, D))
    return z
